# baseline (speedup 1.0000x reference)
import functools

import jax
import jax.numpy as jnp
from jax import lax
from jax.experimental import pallas as pl
from jax.experimental.pallas import tpu as pltpu
from jax.experimental.pallas import tpu_sc as plsc

D = 14336
NROW = 128
B_TOTAL = 4096
NC, NS = 2, 16
NW = NC * NS
B_PER_W = B_TOTAL // NW
INFLIGHT = 16
DMA_SHARE = 10

_mesh = plsc.VectorSubcoreMesh(core_axis_name="c", subcore_axis_name="s")


@functools.partial(
    pl.kernel,
    mesh=_mesh,
    out_type=jax.ShapeDtypeStruct((B_TOTAL, D), jnp.float32),
    scratch_types=[
        pltpu.VMEM((B_PER_W,), jnp.int32),
        pltpu.VMEM((D // 2,), jnp.float32),
        pltpu.VMEM_SHARED((NROW, D), jnp.float32),
        pltpu.SemaphoreType.DMA,
        pltpu.SemaphoreType.DMA,
        pltpu.SemaphoreType.DMA,
    ],
)
def _gather(table_hbm, idx_hbm, out_hbm, idx_v, buf_v, table_sh,
            sem_a, sem_b, sem_l):
    cid = lax.axis_index("c")
    sid = lax.axis_index("s")
    wid = sid * NC + cid
    base = wid * B_PER_W
    rows_per_sub = NROW // NS
    pltpu.sync_copy(
        table_hbm.at[pl.ds(sid * rows_per_sub, rows_per_sub)],
        table_sh.at[pl.ds(sid * rows_per_sub, rows_per_sub)],
    )
    pltpu.sync_copy(idx_hbm.at[pl.ds(base, B_PER_W)], idx_v)
    plsc.subcore_barrier()
    H = D // 2
    copies = []
    for g in range(B_PER_W // 16):
        vec = idx_v[pl.ds(g * 16, 16)]
        for j in range(16):
            c = g * 16 + j
            s = vec[j]
            if c % 16 >= DMA_SHARE:
                for h in range(2):
                    pltpu.sync_copy(table_sh.at[s, pl.ds(h * H, H)], buf_v)
                    pltpu.sync_copy(buf_v, out_hbm.at[base + c, pl.ds(h * H, H)])
            else:
                if len(copies) >= INFLIGHT:
                    copies.pop(0).wait()
                copies.append(
                    pltpu.async_copy(
                        table_sh.at[s], out_hbm.at[base + c], sem_a
                    )
                )
    for cp in copies:
        cp.wait()


def kernel(prefix, table):
    idx = prefix.reshape(-1).astype(jnp.int32)
    out = _gather(table, idx)
    return out.reshape(prefix.shape[0], prefix.shape[1], D)


# speedup vs baseline: 1.0673x; 1.0673x over previous
import functools

import jax
import jax.numpy as jnp
from jax import lax
from jax.experimental import pallas as pl
from jax.experimental.pallas import tpu as pltpu
from jax.experimental.pallas import tpu_sc as plsc

D = 14336
NROW = 128
B_TOTAL = 4096
NC, NS = 2, 16
NW = NC * NS
B_PER_W = B_TOTAL // NW
INFLIGHT = 16
DMA_SHARE = 10

_mesh = plsc.VectorSubcoreMesh(core_axis_name="c", subcore_axis_name="s")


@functools.partial(
    pl.kernel,
    mesh=_mesh,
    out_type=jax.ShapeDtypeStruct((B_TOTAL, D), jnp.float32),
    scratch_types=[
        pltpu.VMEM((B_PER_W,), jnp.int32),
        pltpu.VMEM((6016,), jnp.float32),
        pltpu.VMEM((6016,), jnp.float32),
        pltpu.VMEM_SHARED((NROW, D), jnp.float32),
        pltpu.SemaphoreType.DMA,
        pltpu.SemaphoreType.DMA,
        pltpu.SemaphoreType.DMA,
    ],
)
def _gather(table_hbm, idx_hbm, out_hbm, idx_v, buf0, buf1, table_sh,
            sem_a, sem_b, sem_l):
    cid = lax.axis_index("c")
    sid = lax.axis_index("s")
    wid = sid * NC + cid
    base = wid * B_PER_W
    rows_per_sub = NROW // NS
    pltpu.sync_copy(
        table_hbm.at[pl.ds(sid * rows_per_sub, rows_per_sub)],
        table_sh.at[pl.ds(sid * rows_per_sub, rows_per_sub)],
    )
    pltpu.sync_copy(idx_hbm.at[pl.ds(base, B_PER_W)], idx_v)
    plsc.subcore_barrier()
    CHUNKS = ((0, 6016), (6016, 6016), (12032, 2304))
    bufs = (buf0, buf1)
    copies = []
    outs = []
    pending = None
    qn = 0
    for g in range(B_PER_W // 16):
        vec = idx_v[pl.ds(g * 16, 16)]
        for j in range(16):
            c = g * 16 + j
            s = vec[j]
            if c % 16 >= DMA_SHARE:
                for off, size in CHUNKS:
                    b = qn % 2
                    while outs:
                        outs.pop(0).wait()
                    lc = pltpu.async_copy(
                        table_sh.at[s, pl.ds(off, size)],
                        bufs[b].at[pl.ds(0, size)],
                        sem_l,
                    )
                    if pending is not None:
                        plc, pb, prow, poff, psize = pending
                        plc.wait()
                        outs.append(
                            pltpu.async_copy(
                                bufs[pb].at[pl.ds(0, psize)],
                                out_hbm.at[prow, pl.ds(poff, psize)],
                                sem_b,
                            )
                        )
                    pending = (lc, b, base + c, off, size)
                    qn += 1
            else:
                if len(copies) >= INFLIGHT:
                    copies.pop(0).wait()
                copies.append(
                    pltpu.async_copy(
                        table_sh.at[s], out_hbm.at[base + c], sem_a
                    )
                )
    if pending is not None:
        plc, pb, prow, poff, psize = pending
        plc.wait()
        outs.append(
            pltpu.async_copy(
                bufs[pb].at[pl.ds(0, psize)],
                out_hbm.at[prow, pl.ds(poff, psize)],
                sem_b,
            )
        )
    for cp in outs:
        cp.wait()
    for cp in copies:
        cp.wait()


def kernel(prefix, table):
    idx = prefix.reshape(-1).astype(jnp.int32)
    out = _gather(table, idx)
    return out.reshape(prefix.shape[0], prefix.shape[1], D)
